# baseline (speedup 1.0000x reference)
import functools

import jax
import jax.numpy as jnp
from jax.experimental import pallas as pl

_ROWS_PER_BLOCK = 16

_KS0 = 0
_KS1 = 42
_KS2 = _KS0 ^ _KS1 ^ 0x1BD11BDA
_ROT_A = (13, 15, 26, 6)
_ROT_B = (17, 29, 16, 24)


def _rotl(x, r):
    return (x << jnp.uint32(r)) | (x >> jnp.uint32(32 - r))


def _threefry_bits(flat_idx_u32):
    ks = (jnp.uint32(_KS0), jnp.uint32(_KS1), jnp.uint32(_KS2))
    x0 = jnp.uint32(_KS0)
    x1 = flat_idx_u32 + jnp.uint32(_KS1)
    rots = (_ROT_A, _ROT_B)
    for blk in range(5):
        for r in rots[blk % 2]:
            x0 = x0 + x1
            x1 = _rotl(x1, r)
            x1 = x1 ^ x0
        x0 = x0 + ks[(blk + 1) % 3]
        x1 = x1 + ks[(blk + 2) % 3] + jnp.uint32(blk + 1)
    return x0 ^ x1


def _gumbel_softmax_block(x_ref, o_ref, *, n_cols, rb):
    i = pl.program_id(0)
    row = jax.lax.broadcasted_iota(jnp.int32, (rb, n_cols), 0) + i * rb
    col = jax.lax.broadcasted_iota(jnp.int32, (rb, n_cols), 1)
    flat = (row * n_cols + col).astype(jnp.uint32)
    bits = _threefry_bits(flat)
    fbits = (bits >> jnp.uint32(9)) | jnp.uint32(0x3F800000)
    u = jax.lax.bitcast_convert_type(fbits, jnp.float32) - 1.0
    g = -jnp.log(-jnp.log(u + 1e-10) + 1e-10)
    y = x_ref[...] + g
    m = jnp.max(y, axis=-1, keepdims=True)
    e = jnp.exp(y - m)
    s = jnp.sum(e, axis=-1, keepdims=True)
    o_ref[...] = e / s


def kernel(logits):
    n_rows, n_cols = logits.shape
    rb = _ROWS_PER_BLOCK if n_rows % _ROWS_PER_BLOCK == 0 else 1
    spec = pl.BlockSpec((rb, n_cols), lambda i: (i, 0))
    body = functools.partial(_gumbel_softmax_block, n_cols=n_cols, rb=rb)
    return pl.pallas_call(
        body,
        grid=(n_rows // rb,),
        in_specs=[spec],
        out_specs=spec,
        out_shape=jax.ShapeDtypeStruct(logits.shape, logits.dtype),
    )(logits)


# speedup vs baseline: 1.2567x; 1.2567x over previous
import functools

import jax
import jax.numpy as jnp
from jax.experimental import pallas as pl

_ROWS_PER_BLOCK = 8
_CHUNK = 2048

_KS0 = 0
_KS1 = 42
_KS2 = _KS0 ^ _KS1 ^ 0x1BD11BDA
_ROT_A = (13, 15, 26, 6)
_ROT_B = (17, 29, 16, 24)


def _rotl(x, r):
    return (x << jnp.uint32(r)) | (x >> jnp.uint32(32 - r))


def _threefry_bits(flat_idx_u32):
    ks = (jnp.uint32(_KS0), jnp.uint32(_KS1), jnp.uint32(_KS2))
    x0 = jnp.zeros_like(flat_idx_u32) + jnp.uint32(_KS0)
    x1 = flat_idx_u32 + jnp.uint32(_KS1)
    rots = (_ROT_A, _ROT_B)
    for blk in range(5):
        for r in rots[blk % 2]:
            x0 = x0 + x1
            x1 = _rotl(x1, r)
            x1 = x1 ^ x0
        x0 = x0 + ks[(blk + 1) % 3]
        x1 = x1 + ks[(blk + 2) % 3] + jnp.uint32(blk + 1)
    return x0 ^ x1


def _neg_log_u(flat_idx_i32):
    bits = _threefry_bits(flat_idx_i32.astype(jnp.uint32))
    fbits = (bits >> jnp.uint32(9)) | jnp.uint32(0x3F800000)
    u = jax.lax.bitcast_convert_type(fbits, jnp.float32) - 1.0
    return -jnp.log(u + 1e-10) + 1e-10


def _gumbel_softmax_block(x_ref, o_ref, *, n_cols, rb):
    nfull = n_cols // _CHUNK
    tail = n_cols % _CHUNK
    tail_start = nfull * _CHUNK

    i = pl.program_id(0)
    row = jax.lax.broadcasted_iota(jnp.int32, (rb, _CHUNK), 0) + i * rb
    col = jax.lax.broadcasted_iota(jnp.int32, (rb, _CHUNK), 1)
    flat0 = row * n_cols + col

    def p1(k, carry):
        m, flat = carry
        sl = pl.ds(k * _CHUNK, _CHUNK)
        x = x_ref[:, sl]
        o_ref[:, sl] = _neg_log_u(flat)
        m = jnp.maximum(m, jnp.max(x, axis=-1, keepdims=True))
        return m, flat + _CHUNK

    m0 = jnp.full((rb, 1), -jnp.inf, jnp.float32)
    m, _ = jax.lax.fori_loop(0, nfull, p1, (m0, flat0))
    if tail:
        rowt = jax.lax.broadcasted_iota(jnp.int32, (rb, tail), 0) + i * rb
        colt = jax.lax.broadcasted_iota(jnp.int32, (rb, tail), 1) + tail_start
        xt = x_ref[:, tail_start:]
        o_ref[:, tail_start:] = _neg_log_u(rowt * n_cols + colt)
        m = jnp.maximum(m, jnp.max(xt, axis=-1, keepdims=True))

    def p2(k, s):
        sl = pl.ds(k * _CHUNK, _CHUNK)
        e = jnp.exp(x_ref[:, sl] - m) / o_ref[:, sl]
        o_ref[:, sl] = e
        return s + jnp.sum(e, axis=-1, keepdims=True)

    s = jax.lax.fori_loop(0, nfull, p2, jnp.zeros((rb, 1), jnp.float32))
    if tail:
        e = jnp.exp(x_ref[:, tail_start:] - m) / o_ref[:, tail_start:]
        o_ref[:, tail_start:] = e
        s = s + jnp.sum(e, axis=-1, keepdims=True)

    rinv = 1.0 / s

    def p3(k, _):
        sl = pl.ds(k * _CHUNK, _CHUNK)
        o_ref[:, sl] = o_ref[:, sl] * rinv
        return 0

    jax.lax.fori_loop(0, nfull, p3, 0)
    if tail:
        o_ref[:, tail_start:] = o_ref[:, tail_start:] * rinv


def kernel(logits):
    n_rows, n_cols = logits.shape
    rb = _ROWS_PER_BLOCK if n_rows % _ROWS_PER_BLOCK == 0 else 1
    spec = pl.BlockSpec((rb, n_cols), lambda i: (i, 0))
    body = functools.partial(_gumbel_softmax_block, n_cols=n_cols, rb=rb)
    return pl.pallas_call(
        body,
        grid=(n_rows // rb,),
        in_specs=[spec],
        out_specs=spec,
        out_shape=jax.ShapeDtypeStruct(logits.shape, logits.dtype),
    )(logits)


# speedup vs baseline: 1.2726x; 1.0126x over previous
import functools

import jax
import jax.numpy as jnp
from jax.experimental import pallas as pl

_ROWS_PER_BLOCK = 8
_CHUNK = 8192

_KS0 = 0
_KS1 = 42
_KS2 = _KS0 ^ _KS1 ^ 0x1BD11BDA
_ROT_A = (13, 15, 26, 6)
_ROT_B = (17, 29, 16, 24)


def _rotl(x, r):
    return (x << jnp.uint32(r)) | (x >> jnp.uint32(32 - r))


def _threefry_bits(flat_idx_u32):
    ks = (jnp.uint32(_KS0), jnp.uint32(_KS1), jnp.uint32(_KS2))
    x0 = jnp.zeros_like(flat_idx_u32) + jnp.uint32(_KS0)
    x1 = flat_idx_u32 + jnp.uint32(_KS1)
    rots = (_ROT_A, _ROT_B)
    for blk in range(5):
        for r in rots[blk % 2]:
            x0 = x0 + x1
            x1 = _rotl(x1, r)
            x1 = x1 ^ x0
        x0 = x0 + ks[(blk + 1) % 3]
        x1 = x1 + ks[(blk + 2) % 3] + jnp.uint32(blk + 1)
    return x0 ^ x1


def _neg_log_u(flat_idx_i32):
    bits = _threefry_bits(flat_idx_i32.astype(jnp.uint32))
    fbits = (bits >> jnp.uint32(9)) | jnp.uint32(0x3F800000)
    u = jax.lax.bitcast_convert_type(fbits, jnp.float32) - 1.0
    return -jnp.log(u + 1e-10) + 1e-10


def _gumbel_softmax_block(x_ref, o_ref, *, n_cols, rb):
    nfull = n_cols // _CHUNK
    tail = n_cols % _CHUNK
    tail_start = nfull * _CHUNK

    i = pl.program_id(0)
    row = jax.lax.broadcasted_iota(jnp.int32, (rb, _CHUNK), 0) + i * rb
    col = jax.lax.broadcasted_iota(jnp.int32, (rb, _CHUNK), 1)
    flat0 = row * n_cols + col

    def p1(k, carry):
        m, flat = carry
        sl = pl.ds(k * _CHUNK, _CHUNK)
        x = x_ref[:, sl]
        o_ref[:, sl] = _neg_log_u(flat)
        m = jnp.maximum(m, jnp.max(x, axis=-1, keepdims=True))
        return m, flat + _CHUNK

    m0 = jnp.full((rb, 1), -jnp.inf, jnp.float32)
    m, _ = jax.lax.fori_loop(0, nfull, p1, (m0, flat0))
    if tail:
        rowt = jax.lax.broadcasted_iota(jnp.int32, (rb, tail), 0) + i * rb
        colt = jax.lax.broadcasted_iota(jnp.int32, (rb, tail), 1) + tail_start
        xt = x_ref[:, tail_start:]
        o_ref[:, tail_start:] = _neg_log_u(rowt * n_cols + colt)
        m = jnp.maximum(m, jnp.max(xt, axis=-1, keepdims=True))

    def p2(k, s):
        sl = pl.ds(k * _CHUNK, _CHUNK)
        e = jnp.exp(x_ref[:, sl] - m) / o_ref[:, sl]
        o_ref[:, sl] = e
        return s + jnp.sum(e, axis=-1, keepdims=True)

    s = jax.lax.fori_loop(0, nfull, p2, jnp.zeros((rb, 1), jnp.float32))
    if tail:
        e = jnp.exp(x_ref[:, tail_start:] - m) / o_ref[:, tail_start:]
        o_ref[:, tail_start:] = e
        s = s + jnp.sum(e, axis=-1, keepdims=True)

    rinv = 1.0 / s

    def p3(k, _):
        sl = pl.ds(k * _CHUNK, _CHUNK)
        o_ref[:, sl] = o_ref[:, sl] * rinv
        return 0

    jax.lax.fori_loop(0, nfull, p3, 0)
    if tail:
        o_ref[:, tail_start:] = o_ref[:, tail_start:] * rinv


def kernel(logits):
    n_rows, n_cols = logits.shape
    rb = _ROWS_PER_BLOCK if n_rows % _ROWS_PER_BLOCK == 0 else 1
    spec = pl.BlockSpec((rb, n_cols), lambda i: (i, 0))
    body = functools.partial(_gumbel_softmax_block, n_cols=n_cols, rb=rb)
    return pl.pallas_call(
        body,
        grid=(n_rows // rb,),
        in_specs=[spec],
        out_specs=spec,
        out_shape=jax.ShapeDtypeStruct(logits.shape, logits.dtype),
    )(logits)
